# Initial kernel scaffold; baseline (speedup 1.0000x reference)
import jax
import jax.numpy as jnp
from jax.experimental import pallas as pl


def kernel(x, q, story_table, query_table, story_W, story_b, query_W, query_b):
    raise NotImplementedError("write your pallas kernel here")


# speedup vs baseline: 7.2618x; 7.2618x over previous
import functools

import jax
import jax.numpy as jnp
from jax import lax
from jax.experimental import pallas as pl
from jax.experimental.pallas import tpu as pltpu
from jax.experimental.pallas import tpu_sc as plsc

NUM_WORDS = 100000
EMB_DIM = 64
STORY_WIDTH = 50
QUERY_WIDTH = 20
BS = 1024
N_WORD = 20

NC = 2
NS = 16
NW = NC * NS
LANES = 16

B_PER_W = BS // NW
S_IDX = STORY_WIDTH * N_WORD
Q_PER_W = B_PER_W * QUERY_WIDTH

def _chunks(total, step=128):
    out = []
    off = 0
    while off < total:
        out.append((off, min(step, total - off)))
        off += step
    return out

S_CHUNKS = _chunks(S_IDX)
Q_CHUNKS = _chunks(Q_PER_W)


def _sc_body(x_hbm, q_hbm, stab_hbm, qtab_hbm, semb_hbm, qemb_hbm,
             idx_v, rows_v, semb_v, qidx_v, qrows_v, sem):
    cid = lax.axis_index("c")
    sid = lax.axis_index("s")
    wid = sid * NC + cid
    b0 = wid * B_PER_W

    pltpu.sync_copy(q_hbm.at[pl.ds(wid * Q_PER_W, Q_PER_W)], qidx_v)
    qcopies = [
        pltpu.async_copy(qtab_hbm.at[qidx_v.at[pl.ds(off, sz)]],
                         qrows_v.at[pl.ds(off, sz)], sem)
        for off, sz in Q_CHUNKS
    ]
    for c in qcopies:
        c.wait()
    pltpu.sync_copy(qrows_v, qemb_hbm.at[pl.ds(wid * Q_PER_W, Q_PER_W)])

    def batch_body(i, carry):
        b = b0 + i
        pltpu.sync_copy(x_hbm.at[b], idx_v)
        copies = [
            pltpu.async_copy(stab_hbm.at[idx_v.at[pl.ds(off, sz)]],
                             rows_v.at[pl.ds(off, sz)], sem)
            for off, sz in S_CHUNKS
        ]
        for c in copies:
            c.wait()

        def seg_body(s, carry2):
            base = s * N_WORD
            for ci in range(EMB_DIM // LANES):
                sl = pl.ds(ci * LANES, LANES)
                acc = rows_v[base, sl]
                for w in range(1, N_WORD):
                    acc = acc + rows_v[base + w, sl]
                semb_v[s, sl] = acc
            return carry2

        lax.fori_loop(0, STORY_WIDTH, seg_body, 0)
        pltpu.sync_copy(semb_v, semb_hbm.at[b])
        return carry

    lax.fori_loop(0, B_PER_W, batch_body, 0)


_sc_gather = pl.kernel(
    _sc_body,
    out_type=(
        jax.ShapeDtypeStruct((BS, STORY_WIDTH, EMB_DIM), jnp.float32),
        jax.ShapeDtypeStruct((BS * QUERY_WIDTH, EMB_DIM), jnp.float32),
    ),
    mesh=plsc.VectorSubcoreMesh(core_axis_name="c", subcore_axis_name="s",
                                num_cores=NC, num_subcores=NS),
    scratch_types=[
        pltpu.VMEM((S_IDX,), jnp.int32),
        pltpu.VMEM((S_IDX, EMB_DIM), jnp.float32),
        pltpu.VMEM((STORY_WIDTH, EMB_DIM), jnp.float32),
        pltpu.VMEM((Q_PER_W,), jnp.int32),
        pltpu.VMEM((Q_PER_W, EMB_DIM), jnp.float32),
        pltpu.SemaphoreType.DMA,
    ],
    compiler_params=pltpu.CompilerParams(use_tc_tiling_on_sc=False),
    name="sc_embedding_gather_sum",
)


TC_BLK = 8


def _tc_body(semb_ref, qemb_ref, sW_ref, sb_ref, qW_ref, qb_ref,
             fx_ref, sx_ref, fq_ref, sq_ref):
    semb = semb_ref[...]
    qemb = qemb_ref[...]
    fx_ref[...] = jnp.sum(semb, axis=1)
    fq_ref[...] = jnp.sum(qemb, axis=1)
    sW = sW_ref[...]
    qW = qW_ref[...]
    sb = sb_ref[...]
    qb = qb_ref[...]
    for i in range(TC_BLK):
        sx_ref[i] = jnp.dot(sW, semb[i], preferred_element_type=jnp.float32) + sb
        sq_ref[i] = jnp.dot(qW, qemb[i], preferred_element_type=jnp.float32) + qb


@functools.partial(jax.jit, static_argnames=())
def _tc_linear(semb, qemb, sW, sb_col, qW, qb_col):
    grid = (BS // TC_BLK,)
    return pl.pallas_call(
        _tc_body,
        grid=grid,
        in_specs=[
            pl.BlockSpec((TC_BLK, STORY_WIDTH, EMB_DIM), lambda i: (i, 0, 0)),
            pl.BlockSpec((TC_BLK, QUERY_WIDTH, EMB_DIM), lambda i: (i, 0, 0)),
            pl.BlockSpec((EMB_DIM, STORY_WIDTH), lambda i: (0, 0)),
            pl.BlockSpec((EMB_DIM, 1), lambda i: (0, 0)),
            pl.BlockSpec((EMB_DIM, QUERY_WIDTH), lambda i: (0, 0)),
            pl.BlockSpec((EMB_DIM, 1), lambda i: (0, 0)),
        ],
        out_specs=[
            pl.BlockSpec((TC_BLK, EMB_DIM), lambda i: (i, 0)),
            pl.BlockSpec((TC_BLK, EMB_DIM, EMB_DIM), lambda i: (i, 0, 0)),
            pl.BlockSpec((TC_BLK, EMB_DIM), lambda i: (i, 0)),
            pl.BlockSpec((TC_BLK, EMB_DIM, EMB_DIM), lambda i: (i, 0, 0)),
        ],
        out_shape=[
            jax.ShapeDtypeStruct((BS, EMB_DIM), jnp.float32),
            jax.ShapeDtypeStruct((BS, EMB_DIM, EMB_DIM), jnp.float32),
            jax.ShapeDtypeStruct((BS, EMB_DIM), jnp.float32),
            jax.ShapeDtypeStruct((BS, EMB_DIM, EMB_DIM), jnp.float32),
        ],
    )(semb, qemb, sW, sb_col, qW, qb_col)


def kernel(x, q, story_table, query_table, story_W, story_b, query_W, query_b):
    x2 = jnp.reshape(x, (BS, S_IDX)).astype(jnp.int32)
    qf = jnp.reshape(q, (BS * QUERY_WIDTH,)).astype(jnp.int32)
    semb, qemb2 = _sc_gather(x2, qf, story_table, query_table)
    qemb = jnp.reshape(qemb2, (BS, QUERY_WIDTH, EMB_DIM))
    fx, sx, fq, sq = _tc_linear(
        semb, qemb,
        story_W, jnp.reshape(story_b, (EMB_DIM, 1)),
        query_W, jnp.reshape(query_b, (EMB_DIM, 1)),
    )
    return (fx, sx, fq, sq)
